# Initial kernel scaffold; baseline (speedup 1.0000x reference)
import jax
import jax.numpy as jnp
from jax.experimental import pallas as pl


def kernel(users, items, adj_indices, adj_values, user_emb, item_emb):
    raise NotImplementedError("write your pallas kernel here")


# speedup vs baseline: 12.3571x; 12.3571x over previous
import functools

import jax
import jax.numpy as jnp
import numpy as np
from jax import lax
from jax.experimental import pallas as pl
from jax.experimental.pallas import tpu as pltpu
from jax.experimental.pallas import tpu_sc as plsc

NU = 25000
NI = 25000
NN = NU + NI
EE = 1600000
DD = 32
BB = 4096
NC = 2
NS = 16
NW = NC * NS

EPW = EE // NW
SUB = 4
IDX = 100
CH = SUB * IDX
NCHUNK = EPW // CH
NP = 50048
RPT = NP // NS
ZR = 184
NB = BB // NW

_mesh = plsc.VectorSubcoreMesh(core_axis_name="c", subcore_axis_name="s")


@functools.partial(
    pl.kernel,
    out_type=jax.ShapeDtypeStruct((NC * NP, DD), jnp.float32),
    mesh=_mesh,
    scratch_types=[
        pltpu.VMEM_SHARED((NP, DD), jnp.float32),
        pltpu.VMEM((SUB, IDX), jnp.int32),
        pltpu.VMEM((SUB, IDX), jnp.int32),
        pltpu.VMEM((CH,), jnp.float32),
        pltpu.VMEM((CH, DD), jnp.float32),
        pltpu.VMEM((ZR, DD), jnp.float32),
        pltpu.SemaphoreType.DMA,
    ],
    compiler_params=pltpu.CompilerParams(use_tc_tiling_on_sc=False),
)
def _edge_pass(col2d, row2d, val_hbm, table, out, acc, colv, rowv, valv,
               msgs, zbuf, sem):
    c = lax.axis_index("c")
    s = lax.axis_index("s")
    w = s * NC + c

    zf = jnp.zeros((16,), jnp.float32)

    @pl.loop(0, ZR)
    def _(i):
        zbuf[i, 0:16] = zf
        zbuf[i, 16:32] = zf

    for i in range(RPT // ZR):
        pltpu.sync_copy(zbuf, acc.at[pl.ds(s * RPT + i * ZR, ZR)])
    plsc.subcore_barrier()

    @pl.loop(0, NCHUNK)
    def _(ci):
        r0 = w * (EPW // IDX) + ci * SUB
        pltpu.sync_copy(col2d.at[pl.ds(r0, SUB)], colv)
        pltpu.sync_copy(row2d.at[pl.ds(r0, SUB)], rowv)
        pltpu.sync_copy(val_hbm.at[pl.ds(w * EPW + ci * CH, CH)], valv)
        cps = [
            pltpu.async_copy(table.at[colv.at[j]],
                             msgs.at[pl.ds(j * IDX, IDX)], sem)
            for j in range(SUB)
        ]
        for cp in cps:
            cp.wait()

        zero16 = lax.iota(jnp.int32, 16) * 0

        @pl.loop(0, CH // 16)
        def _(g):
            vv = valv[pl.ds(g * 16, 16)]
            for e in range(16):
                k = g * 16 + e
                sp = vv.at[zero16 + e].get(mode="promise_in_bounds")
                msgs[k, 0:16] = msgs[k, 0:16] * sp
                msgs[k, 16:32] = msgs[k, 16:32] * sp

        for j in range(SUB):
            pltpu.sync_copy(msgs.at[pl.ds(j * IDX, IDX)],
                            acc.at[rowv.at[j]], add=True)

    plsc.subcore_barrier()
    pltpu.sync_copy(acc.at[pl.ds(s * RPT, RPT)],
                    out.at[pl.ds(c * NP + s * RPT, RPT)])


def _merge_body(p0_ref, p1_ref, o_ref):
    o_ref[...] = p0_ref[...] + p1_ref[...]


def _merge(part):
    p0 = part[:NP]
    p1 = part[NP:]
    return pl.pallas_call(
        _merge_body,
        out_shape=jax.ShapeDtypeStruct((NP, DD), jnp.float32),
        grid=(16,),
        in_specs=[
            pl.BlockSpec((NP // 16, DD), lambda i: (i, 0)),
            pl.BlockSpec((NP // 16, DD), lambda i: (i, 0)),
        ],
        out_specs=pl.BlockSpec((NP // 16, DD), lambda i: (i, 0)),
    )(p0, p1)


@functools.partial(
    pl.kernel,
    out_type=jax.ShapeDtypeStruct((BB,), jnp.float32),
    mesh=_mesh,
    scratch_types=[
        pltpu.VMEM((NB,), jnp.int32),
        pltpu.VMEM((NB,), jnp.int32),
        pltpu.VMEM((NB, DD), jnp.float32),
        pltpu.VMEM((NB, DD), jnp.float32),
        pltpu.VMEM((NB, DD), jnp.float32),
        pltpu.VMEM((NB,), jnp.float32),
        pltpu.SemaphoreType.DMA,
    ],
    compiler_params=pltpu.CompilerParams(use_tc_tiling_on_sc=False,
                                         needs_layout_passes=False),
)
def _final(users, items, t0, t1, t2, p20, p21, gamma, uidx, iidx, usum,
           isum, tmp, gout, sem):
    c = lax.axis_index("c")
    s = lax.axis_index("s")
    w = s * NC + c
    b0 = w * NB
    pltpu.sync_copy(users.at[pl.ds(b0, NB)], uidx)
    pltpu.sync_copy(items.at[pl.ds(b0, NB)], iidx)

    @pl.loop(0, NB // 16)
    def _(i):
        iidx[pl.ds(i * 16, 16)] = iidx[pl.ds(i * 16, 16)] + NU

    pltpu.async_copy(t0.at[uidx], usum, sem).wait()
    pltpu.async_copy(t0.at[iidx], isum, sem).wait()

    def _add_rows(dst, src):
        @pl.loop(0, NB, unroll=4)
        def _(i):
            dst[i, 0:16] = dst[i, 0:16] + src[i, 0:16]
            dst[i, 16:32] = dst[i, 16:32] + src[i, 16:32]

    for tbl in (t1, t2, p20, p21):
        pltpu.async_copy(tbl.at[uidx], tmp, sem).wait()
        _add_rows(usum, tmp)
        pltpu.async_copy(tbl.at[iidx], tmp, sem).wait()
        _add_rows(isum, tmp)

    lanes = lax.iota(jnp.int32, 16)

    @pl.loop(0, NB // 16)
    def _(g):
        gvec = jnp.zeros((16,), jnp.float32)
        for e in range(16):
            b = g * 16 + e
            prod = (usum[b, 0:16] * isum[b, 0:16]
                    + usum[b, 16:32] * isum[b, 16:32])
            gvec = jnp.where(lanes == e, jnp.sum(prod), gvec)
        gout[pl.ds(g * 16, 16)] = gvec * (1.0 / 16.0)

    pltpu.sync_copy(gout, gamma.at[pl.ds(b0, NB)])


def kernel(users, items, adj_indices, adj_values, user_emb, item_emb):
    row = adj_indices[0]
    col = adj_indices[1]
    col2d = col.reshape(EE // IDX, IDX)
    row2d = row.reshape(EE // IDX, IDX)
    table0 = jnp.concatenate([user_emb, item_emb], axis=0)

    part0 = _edge_pass(col2d, row2d, adj_values, table0)
    t1 = _merge(part0)
    part1 = _edge_pass(col2d, row2d, adj_values, t1)
    t2 = _merge(part1)
    part2 = _edge_pass(col2d, row2d, adj_values, t2)

    gamma = _final(users, items, table0, t1, t2, part2[:NP], part2[NP:])
    return gamma


# speedup vs baseline: 18.9583x; 1.5342x over previous
import functools

import jax
import jax.numpy as jnp
import numpy as np
from jax import lax
from jax.experimental import pallas as pl
from jax.experimental.pallas import tpu as pltpu
from jax.experimental.pallas import tpu_sc as plsc

NU = 25000
NI = 25000
NN = NU + NI
EE = 1600000
DD = 32
BB = 4096
NC = 2
NS = 16
NW = NC * NS

EPW = EE // NW
SUB = 4
IDX = 100
CH = SUB * IDX
NCHUNK = EPW // CH
NP = 50048
RPT = NP // NS
ZR = 184
NB = BB // NW

_mesh = plsc.VectorSubcoreMesh(core_axis_name="c", subcore_axis_name="s")


@functools.partial(
    pl.kernel,
    out_type=jax.ShapeDtypeStruct((NC * NP, DD), jnp.float32),
    mesh=_mesh,
    scratch_types=[
        pltpu.VMEM_SHARED((NP, DD), jnp.float32),
        [pltpu.VMEM((SUB, IDX), jnp.int32)] * 2,
        [pltpu.VMEM((SUB, IDX), jnp.int32)] * 2,
        [pltpu.VMEM((CH,), jnp.float32)] * 2,
        [pltpu.VMEM((CH, DD), jnp.float32)] * 2,
        [pltpu.SemaphoreType.DMA] * 2,
        [pltpu.SemaphoreType.DMA] * 2,
    ],
    compiler_params=pltpu.CompilerParams(use_tc_tiling_on_sc=False),
)
def _edge_pass(col2d, row2d, val_hbm, table, zeros_hbm, out, acc, colv,
               rowv, valv, msgs, sem_lin, sem_gat):
    c = lax.axis_index("c")
    s = lax.axis_index("s")
    w = s * NC + c

    pltpu.sync_copy(zeros_hbm, acc.at[pl.ds(s * RPT, RPT)])
    plsc.subcore_barrier()

    def issue_lin(ci, b):
        r0 = w * (EPW // IDX) + ci * SUB
        pltpu.async_copy(col2d.at[pl.ds(r0, SUB)], colv[b], sem_lin[b])
        pltpu.async_copy(row2d.at[pl.ds(r0, SUB)], rowv[b], sem_lin[b])
        pltpu.async_copy(val_hbm.at[pl.ds(w * EPW + ci * CH, CH)], valv[b],
                         sem_lin[b])

    def wait_lin(b):
        pltpu.make_async_copy(col2d.at[pl.ds(0, SUB)], colv[b],
                              sem_lin[b]).wait()
        pltpu.make_async_copy(row2d.at[pl.ds(0, SUB)], rowv[b],
                              sem_lin[b]).wait()
        pltpu.make_async_copy(val_hbm.at[pl.ds(0, CH)], valv[b],
                              sem_lin[b]).wait()

    def issue_gat(b):
        for j in range(SUB):
            pltpu.async_copy(table.at[colv[b].at[j]],
                             msgs[b].at[pl.ds(j * IDX, IDX)], sem_gat[b])

    def wait_gat(b):
        for j in range(SUB):
            pltpu.make_async_copy(table.at[pl.ds(0, IDX)],
                                  msgs[b].at[pl.ds(j * IDX, IDX)],
                                  sem_gat[b]).wait()

    zero16 = lax.iota(jnp.int32, 16) * 0

    def scale(b):
        m = msgs[b]
        va = valv[b]

        @pl.loop(0, CH // 16, unroll=5)
        def _(g):
            vv = va[pl.ds(g * 16, 16)]
            for e in range(16):
                k = g * 16 + e
                sp = vv.at[zero16 + e].get(mode="promise_in_bounds")
                m[k, 0:16] = m[k, 0:16] * sp
                m[k, 16:32] = m[k, 16:32] * sp

    def scatter(b):
        for j in range(SUB):
            pltpu.sync_copy(msgs[b].at[pl.ds(j * IDX, IDX)],
                            acc.at[rowv[b].at[j]], add=True)

    issue_lin(0, 0)
    wait_lin(0)
    issue_gat(0)
    issue_lin(1, 1)

    def steady(ci, b):
        wait_lin(1 - b)
        issue_gat(1 - b)
        wait_gat(b)
        scale(b)
        scatter(b)
        issue_lin(ci + 2, b)

    @pl.loop(0, (NCHUNK - 3) // 2)
    def _(p):
        steady(2 * p, 0)
        steady(2 * p + 1, 1)

    steady(NCHUNK - 3, 0)
    wait_lin(0)
    issue_gat(0)
    wait_gat(1)
    scale(1)
    scatter(1)
    wait_gat(0)
    scale(0)
    scatter(0)

    plsc.subcore_barrier()
    pltpu.sync_copy(acc.at[pl.ds(s * RPT, RPT)],
                    out.at[pl.ds(c * NP + s * RPT, RPT)])


def _merge_body(p0_ref, p1_ref, o_ref):
    o_ref[...] = p0_ref[...] + p1_ref[...]


def _merge(part):
    p0 = part[:NP]
    p1 = part[NP:]
    return pl.pallas_call(
        _merge_body,
        out_shape=jax.ShapeDtypeStruct((NP, DD), jnp.float32),
        grid=(16,),
        in_specs=[
            pl.BlockSpec((NP // 16, DD), lambda i: (i, 0)),
            pl.BlockSpec((NP // 16, DD), lambda i: (i, 0)),
        ],
        out_specs=pl.BlockSpec((NP // 16, DD), lambda i: (i, 0)),
    )(p0, p1)


@functools.partial(
    pl.kernel,
    out_type=jax.ShapeDtypeStruct((BB,), jnp.float32),
    mesh=_mesh,
    scratch_types=[
        pltpu.VMEM((NB,), jnp.int32),
        pltpu.VMEM((NB,), jnp.int32),
        pltpu.VMEM((NB, DD), jnp.float32),
        pltpu.VMEM((NB, DD), jnp.float32),
        pltpu.VMEM((NB, DD), jnp.float32),
        pltpu.VMEM((NB,), jnp.float32),
        pltpu.SemaphoreType.DMA,
    ],
    compiler_params=pltpu.CompilerParams(use_tc_tiling_on_sc=False,
                                         needs_layout_passes=False),
)
def _final(users, items, t0, t1, t2, p20, p21, gamma, uidx, iidx, usum,
           isum, tmp, gout, sem):
    c = lax.axis_index("c")
    s = lax.axis_index("s")
    w = s * NC + c
    b0 = w * NB
    pltpu.sync_copy(users.at[pl.ds(b0, NB)], uidx)
    pltpu.sync_copy(items.at[pl.ds(b0, NB)], iidx)

    @pl.loop(0, NB // 16)
    def _(i):
        iidx[pl.ds(i * 16, 16)] = iidx[pl.ds(i * 16, 16)] + NU

    pltpu.async_copy(t0.at[uidx], usum, sem).wait()
    pltpu.async_copy(t0.at[iidx], isum, sem).wait()

    def _add_rows(dst, src):
        @pl.loop(0, NB, unroll=4)
        def _(i):
            dst[i, 0:16] = dst[i, 0:16] + src[i, 0:16]
            dst[i, 16:32] = dst[i, 16:32] + src[i, 16:32]

    for tbl in (t1, t2, p20, p21):
        pltpu.async_copy(tbl.at[uidx], tmp, sem).wait()
        _add_rows(usum, tmp)
        pltpu.async_copy(tbl.at[iidx], tmp, sem).wait()
        _add_rows(isum, tmp)

    lanes = lax.iota(jnp.int32, 16)

    @pl.loop(0, NB // 16)
    def _(g):
        gvec = jnp.zeros((16,), jnp.float32)
        for e in range(16):
            b = g * 16 + e
            prod = (usum[b, 0:16] * isum[b, 0:16]
                    + usum[b, 16:32] * isum[b, 16:32])
            gvec = jnp.where(lanes == e, jnp.sum(prod), gvec)
        gout[pl.ds(g * 16, 16)] = gvec * (1.0 / 16.0)

    pltpu.sync_copy(gout, gamma.at[pl.ds(b0, NB)])


def kernel(users, items, adj_indices, adj_values, user_emb, item_emb):
    row = adj_indices[0]
    col = adj_indices[1]
    col2d = col.reshape(EE // IDX, IDX)
    row2d = row.reshape(EE // IDX, IDX)
    table0 = jnp.concatenate([user_emb, item_emb], axis=0)
    zeros = jnp.zeros((RPT, DD), jnp.float32)

    part0 = _edge_pass(col2d, row2d, adj_values, table0, zeros)
    t1 = _merge(part0)
    part1 = _edge_pass(col2d, row2d, adj_values, t1, zeros)
    t2 = _merge(part1)
    part2 = _edge_pass(col2d, row2d, adj_values, t2, zeros)

    gamma = _final(users, items, table0, t1, t2, part2[:NP], part2[NP:])
    return gamma


# speedup vs baseline: 29.5239x; 1.5573x over previous
import functools

import jax
import jax.numpy as jnp
import numpy as np
from jax import lax
from jax.experimental import pallas as pl
from jax.experimental.pallas import tpu as pltpu
from jax.experimental.pallas import tpu_sc as plsc

NU = 25000
NI = 25000
NN = NU + NI
EE = 1600000
DD = 32
BB = 4096
NC = 2
NS = 16
NW = NC * NS

EPW = EE // NW
SUB = 4
IDX = 100
CH = SUB * IDX
NCHUNK = EPW // CH
NP = 50048
RPT = NP // NS
ZR = 184
NB = BB // NW

_mesh = plsc.VectorSubcoreMesh(core_axis_name="c", subcore_axis_name="s")


@functools.partial(
    pl.kernel,
    out_type=jax.ShapeDtypeStruct((NC * NP, DD), jnp.float32),
    mesh=_mesh,
    scratch_types=[
        pltpu.VMEM_SHARED((NP, DD), jnp.float32),
        [pltpu.VMEM((SUB, IDX), jnp.int32)] * 2,
        [pltpu.VMEM((SUB, IDX), jnp.int32)] * 3,
        [pltpu.VMEM((CH,), jnp.float32)] * 2,
        [pltpu.VMEM((CH, DD), jnp.float32)] * 2,
        [pltpu.SemaphoreType.DMA] * 2,
        [pltpu.SemaphoreType.DMA] * 2,
        [pltpu.SemaphoreType.DMA] * 2,
    ],
    compiler_params=pltpu.CompilerParams(use_tc_tiling_on_sc=False),
)
def _edge_pass(col2d, row2d, val_hbm, table, zeros_hbm, out, acc, colv,
               rowv, valv, msgs, sem_lin, sem_gat, sem_sct):
    c = lax.axis_index("c")
    s = lax.axis_index("s")
    w = s * NC + c

    pltpu.sync_copy(zeros_hbm, acc.at[pl.ds(s * RPT, RPT)])
    plsc.subcore_barrier()

    def issue_lin(ci, b, r):
        r0 = w * (EPW // IDX) + ci * SUB
        pltpu.async_copy(col2d.at[pl.ds(r0, SUB)], colv[b], sem_lin[b])
        pltpu.async_copy(row2d.at[pl.ds(r0, SUB)], rowv[r], sem_lin[b])
        pltpu.async_copy(val_hbm.at[pl.ds(w * EPW + ci * CH, CH)], valv[b],
                         sem_lin[b])

    def wait_lin(b, r):
        pltpu.make_async_copy(col2d.at[pl.ds(0, SUB)], colv[b],
                              sem_lin[b]).wait()
        pltpu.make_async_copy(row2d.at[pl.ds(0, SUB)], rowv[r],
                              sem_lin[b]).wait()
        pltpu.make_async_copy(val_hbm.at[pl.ds(0, CH)], valv[b],
                              sem_lin[b]).wait()

    def issue_gat(b):
        for j in range(SUB):
            pltpu.async_copy(table.at[colv[b].at[j]],
                             msgs[b].at[pl.ds(j * IDX, IDX)], sem_gat[b])

    def wait_gat(b):
        for j in range(SUB):
            pltpu.make_async_copy(table.at[pl.ds(0, IDX)],
                                  msgs[b].at[pl.ds(j * IDX, IDX)],
                                  sem_gat[b]).wait()

    zero16 = lax.iota(jnp.int32, 16) * 0

    def scale(b):
        m = msgs[b]
        va = valv[b]

        @pl.loop(0, CH // 16, unroll=5)
        def _(g):
            vv = va[pl.ds(g * 16, 16)]
            for e in range(16):
                k = g * 16 + e
                sp = vv.at[zero16 + e].get(mode="promise_in_bounds")
                m[k, 0:16] = m[k, 0:16] * sp
                m[k, 16:32] = m[k, 16:32] * sp

    def issue_sct(b, r):
        for j in range(SUB):
            pltpu.async_copy(msgs[b].at[pl.ds(j * IDX, IDX)],
                             acc.at[rowv[r].at[j]], sem_sct[b], add=True)

    def wait_sct(b):
        for j in range(SUB):
            pltpu.make_async_copy(zeros_hbm.at[pl.ds(0, IDX)],
                                  msgs[b].at[pl.ds(j * IDX, IDX)],
                                  sem_sct[b]).wait()

    def steady(ci, b, r, first=False, do_lin=True):
        wait_lin(1 - b, (r + 1) % 3)
        if not first:
            wait_sct(1 - b)
        issue_gat(1 - b)
        wait_gat(b)
        scale(b)
        issue_sct(b, r)
        if do_lin:
            issue_lin(ci + 2, b, (r + 2) % 3)

    issue_lin(0, 0, 0)
    wait_lin(0, 0)
    issue_gat(0)
    issue_lin(1, 1, 1)

    steady(0, 0, 0, first=True)

    @pl.loop(0, (NCHUNK - 5) // 6)
    def _(p):
        ci = 1 + 6 * p
        for j in range(6):
            steady(ci + j, (1 + j) % 2, (1 + j) % 3)

    steady(NCHUNK - 4, 1, 1)
    steady(NCHUNK - 3, 0, 2)
    steady(NCHUNK - 2, 1, 0, do_lin=False)
    wait_sct(1)
    wait_gat(0)
    scale(0)
    issue_sct(0, 1)
    wait_sct(0)

    plsc.subcore_barrier()
    pltpu.sync_copy(acc.at[pl.ds(s * RPT, RPT)],
                    out.at[pl.ds(c * NP + s * RPT, RPT)])


MRW = NP // NW
MCH = MRW // 4


@functools.partial(
    pl.kernel,
    out_type=jax.ShapeDtypeStruct((NP, DD), jnp.float32),
    mesh=_mesh,
    scratch_types=[
        [pltpu.VMEM((MCH, DD), jnp.float32)] * 2,
        pltpu.SemaphoreType.DMA,
    ],
    compiler_params=pltpu.CompilerParams(use_tc_tiling_on_sc=False),
)
def _merge(part, out, bufs, sem):
    c = lax.axis_index("c")
    s = lax.axis_index("s")
    w = s * NC + c
    a, bb = bufs
    for k in range(4):
        r0 = w * MRW + k * MCH
        cp0 = pltpu.async_copy(part.at[pl.ds(r0, MCH)], a, sem)
        cp1 = pltpu.async_copy(part.at[pl.ds(NP + r0, MCH)], bb, sem)
        cp0.wait()
        cp1.wait()

        @pl.loop(0, MCH, unroll=4)
        def _(i):
            a[i, 0:16] = a[i, 0:16] + bb[i, 0:16]
            a[i, 16:32] = a[i, 16:32] + bb[i, 16:32]

        pltpu.sync_copy(a, out.at[pl.ds(r0, MCH)])


@functools.partial(
    pl.kernel,
    out_type=jax.ShapeDtypeStruct((BB,), jnp.float32),
    mesh=_mesh,
    scratch_types=[
        pltpu.VMEM((NB,), jnp.int32),
        pltpu.VMEM((NB,), jnp.int32),
        pltpu.VMEM((NB, DD), jnp.float32),
        pltpu.VMEM((NB, DD), jnp.float32),
        pltpu.VMEM((NB, DD), jnp.float32),
        pltpu.VMEM((NB,), jnp.float32),
        pltpu.SemaphoreType.DMA,
    ],
    compiler_params=pltpu.CompilerParams(use_tc_tiling_on_sc=False,
                                         needs_layout_passes=False),
)
def _final(users, items, uemb, iemb, t1, t2, part2, gamma, uidx, iidx, usum,
           isum, tmp, gout, sem):
    c = lax.axis_index("c")
    s = lax.axis_index("s")
    w = s * NC + c
    b0 = w * NB
    pltpu.sync_copy(users.at[pl.ds(b0, NB)], uidx)
    pltpu.sync_copy(items.at[pl.ds(b0, NB)], iidx)

    def _shift(idx, off):
        @pl.loop(0, NB // 16)
        def _(i):
            idx[pl.ds(i * 16, 16)] = idx[pl.ds(i * 16, 16)] + off

    def _add_rows(dst, src):
        @pl.loop(0, NB, unroll=4)
        def _(i):
            dst[i, 0:16] = dst[i, 0:16] + src[i, 0:16]
            dst[i, 16:32] = dst[i, 16:32] + src[i, 16:32]

    def _gather_add(tbl, idx, dst):
        pltpu.async_copy(tbl.at[idx], tmp, sem).wait()
        _add_rows(dst, tmp)

    pltpu.async_copy(uemb.at[uidx], usum, sem).wait()
    _gather_add(t1, uidx, usum)
    _gather_add(t2, uidx, usum)
    _gather_add(part2, uidx, usum)
    _shift(uidx, NP)
    _gather_add(part2, uidx, usum)

    pltpu.async_copy(iemb.at[iidx], isum, sem).wait()
    _shift(iidx, NU)
    _gather_add(t1, iidx, isum)
    _gather_add(t2, iidx, isum)
    _gather_add(part2, iidx, isum)
    _shift(iidx, NP)
    _gather_add(part2, iidx, isum)

    lanes = lax.iota(jnp.int32, 16)

    @pl.loop(0, NB // 16)
    def _(g):
        gvec = jnp.zeros((16,), jnp.float32)
        for e in range(16):
            b = g * 16 + e
            prod = (usum[b, 0:16] * isum[b, 0:16]
                    + usum[b, 16:32] * isum[b, 16:32])
            gvec = jnp.where(lanes == e, jnp.sum(prod), gvec)
        gout[pl.ds(g * 16, 16)] = gvec * (1.0 / 16.0)

    pltpu.sync_copy(gout, gamma.at[pl.ds(b0, NB)])


def kernel(users, items, adj_indices, adj_values, user_emb, item_emb):
    row = adj_indices[0]
    col = adj_indices[1]
    col2d = col.reshape(EE // IDX, IDX)
    row2d = row.reshape(EE // IDX, IDX)
    table0 = jnp.concatenate([user_emb, item_emb], axis=0)
    zeros = jnp.zeros((RPT, DD), jnp.float32)

    part0 = _edge_pass(col2d, row2d, adj_values, table0, zeros)
    t1 = _merge(part0)
    part1 = _edge_pass(col2d, row2d, adj_values, t1, zeros)
    t2 = _merge(part1)
    part2 = _edge_pass(col2d, row2d, adj_values, t2, zeros)

    gamma = _final(users, items, user_emb, item_emb, t1, t2, part2)
    return gamma


# speedup vs baseline: 32.5994x; 1.1042x over previous
import functools

import jax
import jax.numpy as jnp
import numpy as np
from jax import lax
from jax.experimental import pallas as pl
from jax.experimental.pallas import tpu as pltpu
from jax.experimental.pallas import tpu_sc as plsc

NU = 25000
NI = 25000
NN = NU + NI
EE = 1600000
DD = 32
BB = 4096
NC = 2
NS = 16
NW = NC * NS

EPW = EE // NW
SUB = 4
IDX = 100
CH = SUB * IDX
NCHUNK = EPW // CH
NP = 50048
RPT = NP // NS
ZR = 184
NB = BB // NW

_mesh = plsc.VectorSubcoreMesh(core_axis_name="c", subcore_axis_name="s")


@functools.partial(
    pl.kernel,
    out_type=jax.ShapeDtypeStruct((NC * NP, DD), jnp.float32),
    mesh=_mesh,
    scratch_types=[
        pltpu.VMEM_SHARED((NP, DD), jnp.float32),
        [pltpu.VMEM((CH,), jnp.int32)] * 2,
        [pltpu.VMEM((CH,), jnp.int32)] * 3,
        [pltpu.VMEM((CH,), jnp.float32)] * 2,
        [pltpu.VMEM((CH, DD), jnp.float32)] * 2,
        [pltpu.SemaphoreType.DMA] * 2,
        [pltpu.SemaphoreType.DMA] * 2,
        [pltpu.SemaphoreType.DMA] * 2,
    ],
    compiler_params=pltpu.CompilerParams(use_tc_tiling_on_sc=False),
)
def _edge_pass(col_hbm, row_hbm, val_hbm, table, zeros_hbm, out, acc, colv,
               rowv, valv, msgs, sem_lin, sem_gat, sem_sct):
    c = lax.axis_index("c")
    s = lax.axis_index("s")
    w = s * NC + c

    pltpu.sync_copy(zeros_hbm, acc.at[pl.ds(s * RPT, RPT)])
    plsc.subcore_barrier()

    def issue_lin(ci, b, r):
        e0 = w * EPW + ci * CH
        pltpu.async_copy(col_hbm.at[pl.ds(e0, CH)], colv[b], sem_lin[b])
        pltpu.async_copy(row_hbm.at[pl.ds(e0, CH)], rowv[r], sem_lin[b])
        pltpu.async_copy(val_hbm.at[pl.ds(e0, CH)], valv[b], sem_lin[b])

    def wait_lin(b, r):
        pltpu.make_async_copy(col_hbm.at[pl.ds(0, CH)], colv[b],
                              sem_lin[b]).wait()
        pltpu.make_async_copy(row_hbm.at[pl.ds(0, CH)], rowv[r],
                              sem_lin[b]).wait()
        pltpu.make_async_copy(val_hbm.at[pl.ds(0, CH)], valv[b],
                              sem_lin[b]).wait()

    def issue_gat(b):
        pltpu.async_copy(table.at[colv[b]], msgs[b], sem_gat[b])

    def wait_gat(b):
        pltpu.make_async_copy(table.at[pl.ds(0, CH)], msgs[b],
                              sem_gat[b]).wait()

    zero16 = lax.iota(jnp.int32, 16) * 0

    def scale(b):
        m = msgs[b]
        va = valv[b]

        @pl.loop(0, CH // 16, unroll=5)
        def _(g):
            vv = va[pl.ds(g * 16, 16)]
            for e in range(16):
                k = g * 16 + e
                sp = vv.at[zero16 + e].get(mode="promise_in_bounds")
                m[k, 0:16] = m[k, 0:16] * sp
                m[k, 16:32] = m[k, 16:32] * sp

    def issue_sct(b, r):
        pltpu.async_copy(msgs[b], acc.at[rowv[r]], sem_sct[b], add=True)

    def wait_sct(b):
        pltpu.make_async_copy(zeros_hbm.at[pl.ds(0, CH)], msgs[b],
                              sem_sct[b]).wait()

    def steady(ci, b, r, first=False, do_lin=True):
        wait_lin(1 - b, (r + 1) % 3)
        if not first:
            wait_sct(1 - b)
        issue_gat(1 - b)
        wait_gat(b)
        scale(b)
        issue_sct(b, r)
        if do_lin:
            issue_lin(ci + 2, b, (r + 2) % 3)

    issue_lin(0, 0, 0)
    wait_lin(0, 0)
    issue_gat(0)
    issue_lin(1, 1, 1)

    steady(0, 0, 0, first=True)

    @pl.loop(0, (NCHUNK - 5) // 6)
    def _(p):
        ci = 1 + 6 * p
        for j in range(6):
            steady(ci + j, (1 + j) % 2, (1 + j) % 3)

    steady(NCHUNK - 4, 1, 1)
    steady(NCHUNK - 3, 0, 2)
    steady(NCHUNK - 2, 1, 0, do_lin=False)
    wait_sct(1)
    wait_gat(0)
    scale(0)
    issue_sct(0, 1)
    wait_sct(0)

    plsc.subcore_barrier()
    pltpu.sync_copy(acc.at[pl.ds(s * RPT, RPT)],
                    out.at[pl.ds(c * NP + s * RPT, RPT)])


MRW = NP // NW
MCH = MRW // 4


@functools.partial(
    pl.kernel,
    out_type=jax.ShapeDtypeStruct((NP, DD), jnp.float32),
    mesh=_mesh,
    scratch_types=[
        [pltpu.VMEM((MCH, DD), jnp.float32)] * 2,
        pltpu.SemaphoreType.DMA,
    ],
    compiler_params=pltpu.CompilerParams(use_tc_tiling_on_sc=False),
)
def _merge(part, out, bufs, sem):
    c = lax.axis_index("c")
    s = lax.axis_index("s")
    w = s * NC + c
    a, bb = bufs
    for k in range(4):
        r0 = w * MRW + k * MCH
        cp0 = pltpu.async_copy(part.at[pl.ds(r0, MCH)], a, sem)
        cp1 = pltpu.async_copy(part.at[pl.ds(NP + r0, MCH)], bb, sem)
        cp0.wait()
        cp1.wait()

        @pl.loop(0, MCH, unroll=4)
        def _(i):
            a[i, 0:16] = a[i, 0:16] + bb[i, 0:16]
            a[i, 16:32] = a[i, 16:32] + bb[i, 16:32]

        pltpu.sync_copy(a, out.at[pl.ds(r0, MCH)])


@functools.partial(
    pl.kernel,
    out_type=jax.ShapeDtypeStruct((BB,), jnp.float32),
    mesh=_mesh,
    scratch_types=[
        pltpu.VMEM((NB,), jnp.int32),
        pltpu.VMEM((NB,), jnp.int32),
        pltpu.VMEM((NB, DD), jnp.float32),
        pltpu.VMEM((NB, DD), jnp.float32),
        pltpu.VMEM((NB, DD), jnp.float32),
        pltpu.VMEM((NB,), jnp.float32),
        pltpu.SemaphoreType.DMA,
    ],
    compiler_params=pltpu.CompilerParams(use_tc_tiling_on_sc=False,
                                         needs_layout_passes=False),
)
def _final(users, items, uemb, iemb, t1, t2, part2, gamma, uidx, iidx, usum,
           isum, tmp, gout, sem):
    c = lax.axis_index("c")
    s = lax.axis_index("s")
    w = s * NC + c
    b0 = w * NB
    pltpu.sync_copy(users.at[pl.ds(b0, NB)], uidx)
    pltpu.sync_copy(items.at[pl.ds(b0, NB)], iidx)

    def _shift(idx, off):
        @pl.loop(0, NB // 16)
        def _(i):
            idx[pl.ds(i * 16, 16)] = idx[pl.ds(i * 16, 16)] + off

    def _add_rows(dst, src):
        @pl.loop(0, NB, unroll=4)
        def _(i):
            dst[i, 0:16] = dst[i, 0:16] + src[i, 0:16]
            dst[i, 16:32] = dst[i, 16:32] + src[i, 16:32]

    def _gather_add(tbl, idx, dst):
        pltpu.async_copy(tbl.at[idx], tmp, sem).wait()
        _add_rows(dst, tmp)

    pltpu.async_copy(uemb.at[uidx], usum, sem).wait()
    _gather_add(t1, uidx, usum)
    _gather_add(t2, uidx, usum)
    _gather_add(part2, uidx, usum)
    _shift(uidx, NP)
    _gather_add(part2, uidx, usum)

    pltpu.async_copy(iemb.at[iidx], isum, sem).wait()
    _shift(iidx, NU)
    _gather_add(t1, iidx, isum)
    _gather_add(t2, iidx, isum)
    _gather_add(part2, iidx, isum)
    _shift(iidx, NP)
    _gather_add(part2, iidx, isum)

    lanes = lax.iota(jnp.int32, 16)

    @pl.loop(0, NB // 16)
    def _(g):
        gvec = jnp.zeros((16,), jnp.float32)
        for e in range(16):
            b = g * 16 + e
            prod = (usum[b, 0:16] * isum[b, 0:16]
                    + usum[b, 16:32] * isum[b, 16:32])
            gvec = jnp.where(lanes == e, jnp.sum(prod), gvec)
        gout[pl.ds(g * 16, 16)] = gvec * (1.0 / 16.0)

    pltpu.sync_copy(gout, gamma.at[pl.ds(b0, NB)])


def kernel(users, items, adj_indices, adj_values, user_emb, item_emb):
    row = adj_indices[0]
    col = adj_indices[1]
    table0 = jnp.concatenate([user_emb, item_emb], axis=0)
    zeros = jnp.zeros((RPT, DD), jnp.float32)

    part0 = _edge_pass(col, row, adj_values, table0, zeros)
    t1 = _merge(part0)
    part1 = _edge_pass(col, row, adj_values, t1, zeros)
    t2 = _merge(part1)
    part2 = _edge_pass(col, row, adj_values, t2, zeros)

    gamma = _final(users, items, user_emb, item_emb, t1, t2, part2)
    return gamma


# speedup vs baseline: 37.0969x; 1.1380x over previous
import functools

import jax
import jax.numpy as jnp
import numpy as np
from jax import lax
from jax.experimental import pallas as pl
from jax.experimental.pallas import tpu as pltpu
from jax.experimental.pallas import tpu_sc as plsc

NU = 25000
NI = 25000
NN = NU + NI
EE = 1600000
DD = 32
BB = 4096
NC = 2
NS = 16
NW = NC * NS

EPW = EE // NW
SUB = 4
IDX = 100
CH = SUB * IDX
NCHUNK = EPW // CH
NP = 50048
RPT = NP // NS
ZR = 184
NB = BB // NW

_mesh = plsc.VectorSubcoreMesh(core_axis_name="c", subcore_axis_name="s")


@functools.partial(
    pl.kernel,
    out_type=jax.ShapeDtypeStruct((NC * NP, DD), jnp.float32),
    mesh=_mesh,
    scratch_types=[
        pltpu.VMEM_SHARED((NP, DD), jnp.float32),
        [pltpu.VMEM((CH,), jnp.int32)] * 2,
        [pltpu.VMEM((CH,), jnp.int32)] * 3,
        [pltpu.VMEM((CH,), jnp.float32)] * 2,
        [pltpu.VMEM((CH, DD), jnp.float32)] * 2,
        [pltpu.SemaphoreType.DMA] * 2,
        [pltpu.SemaphoreType.DMA] * 2,
        [pltpu.SemaphoreType.DMA] * 2,
    ],
    compiler_params=pltpu.CompilerParams(use_tc_tiling_on_sc=False),
)
def _edge_pass(adj_hbm, val_hbm, table, zeros_hbm, out, acc, colv,
               rowv, valv, msgs, sem_lin, sem_gat, sem_sct):
    c = lax.axis_index("c")
    s = lax.axis_index("s")
    w = s * NC + c

    pltpu.sync_copy(zeros_hbm, acc.at[pl.ds(s * RPT, RPT)])
    plsc.subcore_barrier()

    def issue_lin(ci, b, r):
        e0 = w * EPW + ci * CH
        pltpu.async_copy(adj_hbm.at[pl.ds(EE + e0, CH)], colv[b], sem_lin[b])
        pltpu.async_copy(adj_hbm.at[pl.ds(e0, CH)], rowv[r], sem_lin[b])
        pltpu.async_copy(val_hbm.at[pl.ds(e0, CH)], valv[b], sem_lin[b])

    def wait_lin(b, r):
        pltpu.make_async_copy(adj_hbm.at[pl.ds(0, CH)], colv[b],
                              sem_lin[b]).wait()
        pltpu.make_async_copy(adj_hbm.at[pl.ds(0, CH)], rowv[r],
                              sem_lin[b]).wait()
        pltpu.make_async_copy(val_hbm.at[pl.ds(0, CH)], valv[b],
                              sem_lin[b]).wait()

    def issue_gat(b):
        pltpu.async_copy(table.at[colv[b]], msgs[b], sem_gat[b])

    def wait_gat(b):
        pltpu.make_async_copy(table.at[pl.ds(0, CH)], msgs[b],
                              sem_gat[b]).wait()

    zero16 = lax.iota(jnp.int32, 16) * 0

    def scale(b):
        m = msgs[b]
        va = valv[b]

        @plsc.parallel_loop(0, CH // 16, unroll=5)
        def _(g):
            vv = va[pl.ds(g * 16, 16)]
            for e in range(16):
                k = g * 16 + e
                sp = vv.at[zero16 + e].get(mode="promise_in_bounds")
                m[k, 0:16] = m[k, 0:16] * sp
                m[k, 16:32] = m[k, 16:32] * sp

    def issue_sct(b, r):
        pltpu.async_copy(msgs[b], acc.at[rowv[r]], sem_sct[b], add=True)

    def wait_sct(b):
        pltpu.make_async_copy(zeros_hbm.at[pl.ds(0, CH)], msgs[b],
                              sem_sct[b]).wait()

    def steady(ci, b, r, first=False, do_lin=True):
        wait_lin(1 - b, (r + 1) % 3)
        if not first:
            wait_sct(1 - b)
        issue_gat(1 - b)
        wait_gat(b)
        scale(b)
        issue_sct(b, r)
        if do_lin:
            issue_lin(ci + 2, b, (r + 2) % 3)

    issue_lin(0, 0, 0)
    wait_lin(0, 0)
    issue_gat(0)
    issue_lin(1, 1, 1)

    steady(0, 0, 0, first=True)

    @pl.loop(0, (NCHUNK - 5) // 6)
    def _(p):
        ci = 1 + 6 * p
        for j in range(6):
            steady(ci + j, (1 + j) % 2, (1 + j) % 3)

    steady(NCHUNK - 4, 1, 1)
    steady(NCHUNK - 3, 0, 2)
    steady(NCHUNK - 2, 1, 0, do_lin=False)
    wait_sct(1)
    wait_gat(0)
    scale(0)
    issue_sct(0, 1)
    wait_sct(0)

    plsc.subcore_barrier()
    pltpu.sync_copy(acc.at[pl.ds(s * RPT, RPT)],
                    out.at[pl.ds(c * NP + s * RPT, RPT)])


MRW = NP // NW
MCH = MRW // 4


@functools.partial(
    pl.kernel,
    out_type=jax.ShapeDtypeStruct((NP, DD), jnp.float32),
    mesh=_mesh,
    scratch_types=[
        [pltpu.VMEM((MCH, DD), jnp.float32)] * 2,
        pltpu.SemaphoreType.DMA,
    ],
    compiler_params=pltpu.CompilerParams(use_tc_tiling_on_sc=False),
)
def _merge(part, out, bufs, sem):
    c = lax.axis_index("c")
    s = lax.axis_index("s")
    w = s * NC + c
    a, bb = bufs
    for k in range(4):
        r0 = w * MRW + k * MCH
        cp0 = pltpu.async_copy(part.at[pl.ds(r0, MCH)], a, sem)
        cp1 = pltpu.async_copy(part.at[pl.ds(NP + r0, MCH)], bb, sem)
        cp0.wait()
        cp1.wait()

        @pl.loop(0, MCH, unroll=4)
        def _(i):
            a[i, 0:16] = a[i, 0:16] + bb[i, 0:16]
            a[i, 16:32] = a[i, 16:32] + bb[i, 16:32]

        pltpu.sync_copy(a, out.at[pl.ds(r0, MCH)])


@functools.partial(
    pl.kernel,
    out_type=jax.ShapeDtypeStruct((BB,), jnp.float32),
    mesh=_mesh,
    scratch_types=[
        pltpu.VMEM((NB,), jnp.int32),
        pltpu.VMEM((NB,), jnp.int32),
        pltpu.VMEM((NB, DD), jnp.float32),
        pltpu.VMEM((NB, DD), jnp.float32),
        pltpu.VMEM((NB, DD), jnp.float32),
        pltpu.VMEM((NB,), jnp.float32),
        pltpu.SemaphoreType.DMA,
    ],
    compiler_params=pltpu.CompilerParams(use_tc_tiling_on_sc=False,
                                         needs_layout_passes=False),
)
def _final(users, items, uemb, iemb, t1, t2, part2, gamma, uidx, iidx, usum,
           isum, tmp, gout, sem):
    c = lax.axis_index("c")
    s = lax.axis_index("s")
    w = s * NC + c
    b0 = w * NB
    pltpu.sync_copy(users.at[pl.ds(b0, NB)], uidx)
    pltpu.sync_copy(items.at[pl.ds(b0, NB)], iidx)

    def _shift(idx, off):
        @pl.loop(0, NB // 16)
        def _(i):
            idx[pl.ds(i * 16, 16)] = idx[pl.ds(i * 16, 16)] + off

    def _add_rows(dst, src):
        @pl.loop(0, NB, unroll=4)
        def _(i):
            dst[i, 0:16] = dst[i, 0:16] + src[i, 0:16]
            dst[i, 16:32] = dst[i, 16:32] + src[i, 16:32]

    def _gather_add(tbl, idx, dst):
        pltpu.async_copy(tbl.at[idx], tmp, sem).wait()
        _add_rows(dst, tmp)

    pltpu.async_copy(uemb.at[uidx], usum, sem).wait()
    _gather_add(t1, uidx, usum)
    _gather_add(t2, uidx, usum)
    _gather_add(part2, uidx, usum)
    _shift(uidx, NP)
    _gather_add(part2, uidx, usum)

    pltpu.async_copy(iemb.at[iidx], isum, sem).wait()
    _shift(iidx, NU)
    _gather_add(t1, iidx, isum)
    _gather_add(t2, iidx, isum)
    _gather_add(part2, iidx, isum)
    _shift(iidx, NP)
    _gather_add(part2, iidx, isum)

    lanes = lax.iota(jnp.int32, 16)

    @pl.loop(0, NB // 16)
    def _(g):
        gvec = jnp.zeros((16,), jnp.float32)
        for e in range(16):
            b = g * 16 + e
            prod = (usum[b, 0:16] * isum[b, 0:16]
                    + usum[b, 16:32] * isum[b, 16:32])
            gvec = jnp.where(lanes == e, jnp.sum(prod), gvec)
        gout[pl.ds(g * 16, 16)] = gvec * (1.0 / 16.0)

    pltpu.sync_copy(gout, gamma.at[pl.ds(b0, NB)])


def kernel(users, items, adj_indices, adj_values, user_emb, item_emb):
    adjflat = adj_indices.reshape(2 * EE)
    table0 = jnp.concatenate([user_emb, item_emb], axis=0)
    zeros = jnp.zeros((RPT, DD), jnp.float32)

    part0 = _edge_pass(adjflat, adj_values, table0, zeros)
    t1 = _merge(part0)
    part1 = _edge_pass(adjflat, adj_values, t1, zeros)
    t2 = _merge(part1)
    part2 = _edge_pass(adjflat, adj_values, t2, zeros)

    gamma = _final(users, items, user_emb, item_emb, t1, t2, part2)
    return gamma
